# baseline (speedup 1.0000x reference)
import functools

import jax
import jax.numpy as jnp
from jax import lax
from jax.experimental import pallas as pl
from jax.experimental.pallas import tpu as pltpu
from jax.experimental.pallas import tpu_sc as plsc

NUM_WORKERS = 32
P_TOTAL = 4096
NB = 50
D = 64
P_PER_W = P_TOTAL // NUM_WORKERS
SCALE_F = 8.0


def _emb_body(table, idx, out, idx_v, f0, f1, ob0, ob1, g0, g1, o0, o1):
    w = lax.axis_index("s") * 2 + lax.axis_index("c")

    pltpu.sync_copy(idx.at[w], idx_v)

    fbuf = [f0, f1]
    obuf = [ob0, ob1]
    gsem = [g0, g1]
    osem = [o0, o1]

    ev = lax.iota(jnp.int32, 16) * 2
    rows = [(32 * pb + ev, 32 * pb + ev + 1) for pb in range(4)]

    def start_gather(j, b):
        pltpu.async_copy(table.at[idx_v.at[j]], fbuf[b], gsem[b])

    def wait_gather(j, b):
        pltpu.make_async_copy(table.at[idx_v.at[j]], fbuf[b], gsem[b]).wait()

    def start_out(j, b):
        pltpu.async_copy(obuf[b], out.at[j, :, pl.ds(w * P_PER_W, P_PER_W)],
                         osem[b])

    def wait_out(j, b):
        pltpu.make_async_copy(obuf[b],
                              out.at[j, :, pl.ds(w * P_PER_W, P_PER_W)],
                              osem[b]).wait()

    def compute_chunk(b):
        src, dst = fbuf[b], obuf[b]

        def col_body(j, cj):
            for pb in range(4):
                re, ro = rows[pb]
                a = plsc.load_gather(src, [re, cj]) * SCALE_F
                c = plsc.load_gather(src, [ro, cj]) * SCALE_F
                p = plsc.pack(a, c, format=plsc.PackFormat.INTERLEAVED)
                dst[j, pl.ds(32 * pb, 32)] = p
            return cj + 1

        lax.fori_loop(0, D, col_body, jnp.zeros((16,), jnp.int32), unroll=8)

    start_gather(0, 0)
    start_gather(1, 1)
    for j in (0, 1):
        wait_gather(j, j)
        compute_chunk(j)
        start_gather(j + 2, j)
        start_out(j, j)

    def pair_body(i, _):
        for parity in range(2):
            j = 2 * i + 2 + parity
            b = parity
            wait_gather(j, b)
            wait_out(j - 2, b)
            compute_chunk(b)
            start_gather(j + 2, b)
            start_out(j, b)
        return 0

    lax.fori_loop(0, 23, pair_body, 0)

    for j, b in ((48, 0), (49, 1)):
        wait_gather(j, b)
        wait_out(j - 2, b)
        compute_chunk(b)
        start_out(j, b)
    wait_out(48, 0)
    wait_out(49, 1)


_emb = functools.partial(
    pl.kernel,
    out_type=jax.ShapeDtypeStruct((NB, D, P_TOTAL), jnp.bfloat16),
    mesh=plsc.VectorSubcoreMesh(core_axis_name="c", subcore_axis_name="s"),
    scratch_types=[
        pltpu.VMEM((NB, P_PER_W), jnp.int32),
        pltpu.VMEM((P_PER_W, D), jnp.float32),
        pltpu.VMEM((P_PER_W, D), jnp.float32),
        pltpu.VMEM((D, P_PER_W), jnp.bfloat16),
        pltpu.VMEM((D, P_PER_W), jnp.bfloat16),
        pltpu.SemaphoreType.DMA,
        pltpu.SemaphoreType.DMA,
        pltpu.SemaphoreType.DMA,
        pltpu.SemaphoreType.DMA,
    ],
    compiler_params=pltpu.CompilerParams(
        needs_layout_passes=False,
        use_tc_tiling_on_sc=False,
    ),
)(_emb_body)


def kernel(input, weight):
    idx = jnp.reshape(input, (NUM_WORKERS, P_PER_W, NB))
    idx = jnp.transpose(idx, (0, 2, 1))
    out = _emb(weight, idx)
    return jnp.transpose(out, (2, 0, 1))


# speedup vs baseline: 1.1288x; 1.1288x over previous
import functools

import jax
import jax.numpy as jnp
from jax import lax
from jax.experimental import pallas as pl
from jax.experimental.pallas import tpu as pltpu
from jax.experimental.pallas import tpu_sc as plsc

NUM_WORKERS = 32
P_TOTAL = 4096
NB = 50
D = 64
P_PER_W = P_TOTAL // NUM_WORKERS
SCALE_F = 8.0


def _emb_body(table, idx, out, idx_v, f0, f1, tb, ob0, ob1, g0, g1, o0, o1):
    w = lax.axis_index("s") * 2 + lax.axis_index("c")

    pltpu.sync_copy(idx.at[w], idx_v)

    fbuf = [f0, f1]
    obuf = [ob0, ob1]
    gsem = [g0, g1]
    osem = [o0, o1]

    jv = [(lax.iota(jnp.int32, 16) + 16 * jb) * 131 for jb in range(4)]

    def start_gather(j, b):
        pltpu.async_copy(table.at[idx_v.at[j]], fbuf[b], gsem[b])

    def wait_gather(j, b):
        pltpu.make_async_copy(table.at[idx_v.at[j]], fbuf[b], gsem[b]).wait()

    def start_out(j, b):
        pltpu.async_copy(obuf[b], out.at[j, :, pl.ds(w * P_PER_W, P_PER_W)],
                         osem[b])

    def wait_out(j, b):
        pltpu.make_async_copy(obuf[b],
                              out.at[j, :, pl.ds(w * P_PER_W, P_PER_W)],
                              osem[b]).wait()

    def compute_chunk(b):
        src, dst = fbuf[b], obuf[b]

        def row_body(p, _):
            pm = lax.rem(p, 32)
            slot = p - pm + lax.rem(pm, 2) * 16 + lax.div(pm, 2)
            sv = jnp.full((16,), slot, dtype=jnp.int32)
            for jb in range(4):
                v = src[p, pl.ds(16 * jb, 16)] * SCALE_F
                plsc.store_scatter(tb, [jv[jb] + sv], v)
            return 0

        lax.fori_loop(0, P_PER_W, row_body, 0, unroll=4)

        def col_body(j, _):
            for pb in range(4):
                a = tb[pl.ds(j * 131 + 32 * pb, 16)]
                c = tb[pl.ds(j * 131 + 32 * pb + 16, 16)]
                p = plsc.pack(a, c, format=plsc.PackFormat.INTERLEAVED)
                dst[j, pl.ds(32 * pb, 32)] = p
            return 0

        lax.fori_loop(0, D, col_body, 0, unroll=4)

    start_gather(0, 0)
    start_gather(1, 1)
    for j in (0, 1):
        wait_gather(j, j)
        compute_chunk(j)
        start_gather(j + 2, j)
        start_out(j, j)

    def pair_body(i, _):
        for parity in range(2):
            j = 2 * i + 2 + parity
            b = parity
            wait_gather(j, b)
            wait_out(j - 2, b)
            compute_chunk(b)
            start_gather(j + 2, b)
            start_out(j, b)
        return 0

    lax.fori_loop(0, 23, pair_body, 0)

    for j, b in ((48, 0), (49, 1)):
        wait_gather(j, b)
        wait_out(j - 2, b)
        compute_chunk(b)
        start_out(j, b)
    wait_out(48, 0)
    wait_out(49, 1)


_emb = functools.partial(
    pl.kernel,
    out_type=jax.ShapeDtypeStruct((NB, D, P_TOTAL), jnp.bfloat16),
    mesh=plsc.VectorSubcoreMesh(core_axis_name="c", subcore_axis_name="s"),
    scratch_types=[
        pltpu.VMEM((NB, P_PER_W), jnp.int32),
        pltpu.VMEM((P_PER_W, D), jnp.float32),
        pltpu.VMEM((P_PER_W, D), jnp.float32),
        pltpu.VMEM((D * 131,), jnp.float32),
        pltpu.VMEM((D, P_PER_W), jnp.bfloat16),
        pltpu.VMEM((D, P_PER_W), jnp.bfloat16),
        pltpu.SemaphoreType.DMA,
        pltpu.SemaphoreType.DMA,
        pltpu.SemaphoreType.DMA,
        pltpu.SemaphoreType.DMA,
    ],
    compiler_params=pltpu.CompilerParams(
        needs_layout_passes=False,
        use_tc_tiling_on_sc=False,
    ),
)(_emb_body)


def kernel(input, weight):
    idx = jnp.reshape(input, (NUM_WORKERS, P_PER_W, NB))
    idx = jnp.transpose(idx, (0, 2, 1))
    out = _emb(weight, idx)
    return jnp.transpose(out, (2, 0, 1))
